# baseline (speedup 1.0000x reference)
import functools

import jax
import jax.numpy as jnp
from jax import lax
from jax.experimental import pallas as pl
from jax.experimental.pallas import tpu as pltpu
from jax.experimental.pallas import tpu_sc as plsc

N = 10000
D = 128
T = 7
NSEG = N * T
NSEGP = 71680
E = 320000
EPAD = 327680
NW = 32
EW = EPAD // NW
B = 128
CH = 1024
NB = CH // B
NPAD = 10240

_mesh = plsc.VectorSubcoreMesh(core_axis_name="c", subcore_axis_name="s")


def _iota16():
    return lax.broadcasted_iota(jnp.int32, (16,), 0)


def _mm_body(x_ref, w_ref, u_ref):
    u_ref[0] = jnp.dot(x_ref[...], w_ref[0], preferred_element_type=jnp.float32)


def _compute_u(x, w3):
    return pl.pallas_call(
        _mm_body,
        grid=(T, 5),
        in_specs=[
            pl.BlockSpec((N // 5, D), lambda t, j: (j, 0)),
            pl.BlockSpec((1, D, D), lambda t, j: (t, 0, 0)),
        ],
        out_specs=pl.BlockSpec((1, N // 5, D), lambda t, j: (t, j, 0)),
        out_shape=jax.ShapeDtypeStruct((T, N, D), jnp.float32),
    )(x, w3)


@functools.partial(
    pl.kernel,
    out_type=jax.ShapeDtypeStruct((2 * NSEGP,), jnp.float32),
    mesh=_mesh,
    compiler_params=pltpu.CompilerParams(needs_layout_passes=False),
    scratch_types=[
        pltpu.VMEM_SHARED((NSEGP,), jnp.float32),
        pltpu.VMEM((B,), jnp.float32),
        pltpu.VMEM((CH,), jnp.int32),
        pltpu.VMEM((CH,), jnp.int32),
        pltpu.VMEM((CH,), jnp.int32),
        pltpu.VMEM((2240,), jnp.float32),
        pltpu.SemaphoreType.DMA,
    ],
)
def _sc_counts(rowp, typp, cnt_out, counts_sp, ones, rowc, typc, segc,
               zchunk, semc):
    cid = lax.axis_index("c")
    sid = lax.axis_index("s")
    wid = cid * 16 + sid
    one_v = jnp.full((16,), 1.0, jnp.float32)
    zero_v = jnp.zeros((16,), jnp.float32)

    def fill(r, _):
        ones[pl.ds(r * 16, 16)] = one_v
        return 0
    lax.fori_loop(0, B // 16, fill, 0)

    def zfill(r, _):
        zchunk[pl.ds(r * 16, 16)] = zero_v
        return 0
    lax.fori_loop(0, 140, zfill, 0)

    def zslice(q, _):
        pltpu.sync_copy(zchunk, counts_sp.at[pl.ds(sid * 4480 + q * 2240, 2240)])
        return 0
    lax.fori_loop(0, 2, zslice, 0)
    plsc.subcore_barrier()

    def chunk(c, _):
        base = wid * EW + c * CH
        pltpu.sync_copy(rowp.at[pl.ds(base, CH)], rowc)
        pltpu.sync_copy(typp.at[pl.ds(base, CH)], typc)

        def group(g, _):
            sl = pl.ds(g * 16, 16)
            segc[sl] = rowc[sl] * 7 + typc[sl]
            return 0
        lax.fori_loop(0, CH // 16, group, 0)
        for s in range(NB):
            pltpu.async_copy(
                ones, counts_sp.at[segc.at[pl.ds(s * B, B)]], semc, add=True)
        for s in range(NB):
            pltpu.make_async_copy(
                ones, counts_sp.at[segc.at[pl.ds(0, B)]], semc).wait()
        return 0
    lax.fori_loop(0, EW // CH, chunk, 0)
    plsc.subcore_barrier()

    def rb(q, _):
        r0 = sid * 4480 + q * 2240
        pltpu.sync_copy(counts_sp.at[pl.ds(r0, 2240)], zchunk)
        pltpu.sync_copy(zchunk, cnt_out.at[pl.ds(cid * NSEGP + r0, 2240)])
        return 0
    lax.fori_loop(0, 2, rb, 0)


def _inv_body(c_ref, o_ref):
    c = c_ref[0] + c_ref[1]
    o_ref[...] = jnp.where(c > 0.0, 1.0 / c, 0.0)


def _compute_inv(cpart):
    return pl.pallas_call(
        _inv_body,
        out_shape=jax.ShapeDtypeStruct((NSEGP // 128, 128), jnp.float32),
    )(cpart.reshape(2, NSEGP // 128, 128))


@functools.partial(
    pl.kernel,
    out_type=jax.ShapeDtypeStruct((2, NPAD, D), jnp.float32),
    mesh=_mesh,
    compiler_params=pltpu.CompilerParams(needs_layout_passes=False),
    scratch_types=[
        pltpu.VMEM_SHARED((NPAD, D), jnp.float32),
        pltpu.VMEM((CH,), jnp.int32),
        pltpu.VMEM((CH,), jnp.int32),
        pltpu.VMEM((CH,), jnp.int32),
        pltpu.VMEM((CH,), jnp.int32),
        pltpu.VMEM((CH,), jnp.int32),
        pltpu.VMEM((B,), jnp.int32),
        pltpu.VMEM((B,), jnp.int32),
        pltpu.VMEM((B,), jnp.float32),
        pltpu.VMEM((B,), jnp.float32),
        pltpu.VMEM((B, D), jnp.float32),
        pltpu.VMEM((B, D), jnp.float32),
        pltpu.SemaphoreType.DMA,
        pltpu.SemaphoreType.DMA,
        pltpu.SemaphoreType.DMA,
        pltpu.SemaphoreType.DMA,
        pltpu.SemaphoreType.DMA,
        pltpu.SemaphoreType.DMA,
        pltpu.SemaphoreType.DMA,
        pltpu.SemaphoreType.DMA,
    ],
)
def _sc_main(rowp, colp, typp, u2d, invh, out, acc_sp, rowc, colc, typc,
             segc, gixc, dstA, dstB, invA, invB, ubufA, ubufB,
             semuA, semvA, semiA, semaA, semuB, semvB, semiB, semaB):
    cid = lax.axis_index("c")
    sid = lax.axis_index("s")
    wid = cid * 16 + sid
    zrow = jnp.zeros((16,), jnp.float32)

    def zub(r, _):
        for k in range(8):
            ubufA[r, pl.ds(k * 16, 16)] = zrow
        return 0
    lax.fori_loop(0, B, zub, 0)

    def zacc(q, _):
        pltpu.sync_copy(ubufA, acc_sp.at[pl.ds(sid * 640 + q * B, B)])
        return 0
    lax.fori_loop(0, 5, zacc, 0)
    plsc.subcore_barrier()

    H = B // 2

    def issue(s, ubuf_, invb_, dstb_, semu, semv, semi):
        o = s * B
        pltpu.async_copy(
            u2d.at[gixc.at[pl.ds(o, H)]], ubuf_.at[pl.ds(0, H)], semu)
        pltpu.async_copy(
            u2d.at[gixc.at[pl.ds(o + H, H)]], ubuf_.at[pl.ds(H, H)], semv)
        pltpu.async_copy(invh.at[segc.at[pl.ds(o, B)]], invb_, semi)
        for p in range(8):
            dv = rowc[pl.ds(o + p * 16, 16)]
            dstb_[pl.ds(p * 16, 16)] = dv

    def wait_gather(ubuf_, invb_, semu, semv, semi):
        pltpu.make_async_copy(
            u2d.at[gixc.at[pl.ds(0, H)]], ubuf_.at[pl.ds(0, H)], semu).wait()
        pltpu.make_async_copy(
            u2d.at[gixc.at[pl.ds(0, H)]], ubuf_.at[pl.ds(H, H)], semv).wait()
        pltpu.make_async_copy(invh.at[segc.at[pl.ds(0, B)]], invb_, semi).wait()

    def scale(ubuf_, invb_):
        def e_body(e, _):
            sv = plsc.load_gather(invb_, [jnp.full((16,), e, jnp.int32)])
            for k in range(8):
                sl = pl.ds(k * 16, 16)
                ubuf_[e, sl] = ubuf_[e, sl] * sv
            return 0
        lax.fori_loop(0, B, e_body, 0)

    def issue_add(ubuf_, dstb_, sema):
        pltpu.async_copy(ubuf_, acc_sp.at[dstb_], sema, add=True)

    def wait_add(ubuf_, dstb_, sema):
        pltpu.make_async_copy(ubuf_, acc_sp.at[dstb_], sema).wait()

    def chunk(c, _):
        base = wid * EW + c * CH
        pltpu.sync_copy(rowp.at[pl.ds(base, CH)], rowc)
        pltpu.sync_copy(colp.at[pl.ds(base, CH)], colc)
        pltpu.sync_copy(typp.at[pl.ds(base, CH)], typc)

        def group(g, _):
            sl = pl.ds(g * 16, 16)
            r16 = rowc[sl]
            t16 = typc[sl]
            segc[sl] = r16 * 7 + t16
            gixc[sl] = t16 * N + colc[sl]
            return 0
        lax.fori_loop(0, CH // 16, group, 0)

        issue(0, ubufA, invA, dstA, semuA, semvA, semiA)
        issue(1, ubufB, invB, dstB, semuB, semvB, semiB)

        def pair(q, _):
            wait_gather(ubufA, invA, semuA, semvA, semiA)
            scale(ubufA, invA)
            issue_add(ubufA, dstA, semaA)
            wait_gather(ubufB, invB, semuB, semvB, semiB)
            scale(ubufB, invB)
            issue_add(ubufB, dstB, semaB)
            wait_add(ubufA, dstA, semaA)
            issue(2 * q + 2, ubufA, invA, dstA, semuA, semvA, semiA)
            wait_add(ubufB, dstB, semaB)
            issue(2 * q + 3, ubufB, invB, dstB, semuB, semvB, semiB)
            return 0
        lax.fori_loop(0, NB // 2 - 1, pair, 0)

        wait_gather(ubufA, invA, semuA, semvA, semiA)
        scale(ubufA, invA)
        issue_add(ubufA, dstA, semaA)
        wait_gather(ubufB, invB, semuB, semvB, semiB)
        scale(ubufB, invB)
        issue_add(ubufB, dstB, semaB)
        wait_add(ubufA, dstA, semaA)
        wait_add(ubufB, dstB, semaB)
        return 0
    lax.fori_loop(0, EW // CH, chunk, 0)
    plsc.subcore_barrier()

    def wout(q, _):
        r0 = sid * 640 + q * B
        pltpu.sync_copy(acc_sp.at[pl.ds(r0, B)], out.at[cid, pl.ds(r0, B)])
        return 0
    lax.fori_loop(0, 5, wout, 0)


def _add_body(p_ref, o_ref):
    o_ref[...] = p_ref[0] + p_ref[1]


def _combine(p):
    return pl.pallas_call(
        _add_body,
        grid=(5,),
        in_specs=[pl.BlockSpec((2, N // 5, D), lambda j: (0, j, 0))],
        out_specs=pl.BlockSpec((N // 5, D), lambda j: (j, 0)),
        out_shape=jax.ShapeDtypeStruct((N, D), jnp.float32),
    )(p)


def kernel(x, edge_index, edge_type, weights):
    row = edge_index[0].astype(jnp.int32)
    col = edge_index[1].astype(jnp.int32)
    typ = edge_type.astype(jnp.int32)
    pad = EPAD - E
    rowp = jnp.concatenate([row, jnp.full((pad,), N, jnp.int32)])
    colp = jnp.concatenate([col, jnp.zeros((pad,), jnp.int32)])
    typp = jnp.concatenate([typ, jnp.zeros((pad,), jnp.int32)])
    w3 = weights.reshape(T, D, D)

    u2d = _compute_u(x, w3).reshape(NSEG, D)
    cpart = _sc_counts(rowp, typp)
    inv = _compute_inv(cpart).reshape(NSEGP)
    part = _sc_main(rowp, colp, typp, u2d, inv)
    return _combine(part[:, :N, :])


# speedup vs baseline: 1.1556x; 1.1556x over previous
import functools

import jax
import jax.numpy as jnp
from jax import lax
from jax.experimental import pallas as pl
from jax.experimental.pallas import tpu as pltpu
from jax.experimental.pallas import tpu_sc as plsc

N = 10000
D = 128
T = 7
NSEG = N * T
NSEGP = 71680
E = 320000
EPAD = 327680
NW = 32
EW = EPAD // NW
B = 128
CH = 1024
NB = CH // B
NPAD = 10240
EW0 = 14336
EW1 = 6144

_mesh = plsc.VectorSubcoreMesh(core_axis_name="c", subcore_axis_name="s")


def _iota16():
    return lax.broadcasted_iota(jnp.int32, (16,), 0)


def _mm_body(x_ref, w_ref, u_ref):
    u_ref[0] = jnp.dot(x_ref[...], w_ref[0], preferred_element_type=jnp.float32)


def _compute_u(x, w3):
    return pl.pallas_call(
        _mm_body,
        grid=(T, 5),
        in_specs=[
            pl.BlockSpec((N // 5, D), lambda t, j: (j, 0)),
            pl.BlockSpec((1, D, D), lambda t, j: (t, 0, 0)),
        ],
        out_specs=pl.BlockSpec((1, N // 5, D), lambda t, j: (t, j, 0)),
        out_shape=jax.ShapeDtypeStruct((T, N, D), jnp.float32),
    )(x, w3)


@functools.partial(
    pl.kernel,
    out_type=jax.ShapeDtypeStruct((2 * NSEGP,), jnp.float32),
    mesh=_mesh,
    compiler_params=pltpu.CompilerParams(needs_layout_passes=False),
    scratch_types=[
        pltpu.VMEM_SHARED((NSEGP,), jnp.float32),
        pltpu.VMEM((B,), jnp.float32),
        pltpu.VMEM((CH,), jnp.int32),
        pltpu.VMEM((CH,), jnp.int32),
        pltpu.VMEM((CH,), jnp.int32),
        pltpu.VMEM((2240,), jnp.float32),
        pltpu.SemaphoreType.DMA,
    ],
)
def _sc_counts(rowp, typp, cnt_out, counts_sp, ones, rowc, typc, segc,
               zchunk, semc):
    cid = lax.axis_index("c")
    sid = lax.axis_index("s")
    wid = cid * 16 + sid
    one_v = jnp.full((16,), 1.0, jnp.float32)
    zero_v = jnp.zeros((16,), jnp.float32)

    def fill(r, _):
        ones[pl.ds(r * 16, 16)] = one_v
        return 0
    lax.fori_loop(0, B // 16, fill, 0)

    def zfill(r, _):
        zchunk[pl.ds(r * 16, 16)] = zero_v
        return 0
    lax.fori_loop(0, 140, zfill, 0)

    def zslice(q, _):
        pltpu.sync_copy(zchunk, counts_sp.at[pl.ds(sid * 4480 + q * 2240, 2240)])
        return 0
    lax.fori_loop(0, 2, zslice, 0)
    plsc.subcore_barrier()

    def chunk(c, _):
        base = wid * EW + c * CH
        pltpu.sync_copy(rowp.at[pl.ds(base, CH)], rowc)
        pltpu.sync_copy(typp.at[pl.ds(base, CH)], typc)

        def group(g, _):
            sl = pl.ds(g * 16, 16)
            segc[sl] = rowc[sl] * 7 + typc[sl]
            return 0
        lax.fori_loop(0, CH // 16, group, 0)
        for s in range(NB):
            pltpu.async_copy(
                ones, counts_sp.at[segc.at[pl.ds(s * B, B)]], semc, add=True)
        for s in range(NB):
            pltpu.make_async_copy(
                ones, counts_sp.at[segc.at[pl.ds(0, B)]], semc).wait()
        return 0
    lax.fori_loop(0, EW // CH, chunk, 0)
    plsc.subcore_barrier()

    def rb(q, _):
        r0 = sid * 4480 + q * 2240
        pltpu.sync_copy(counts_sp.at[pl.ds(r0, 2240)], zchunk)
        pltpu.sync_copy(zchunk, cnt_out.at[pl.ds(cid * NSEGP + r0, 2240)])
        return 0
    lax.fori_loop(0, 2, rb, 0)


def _inv_body(c_ref, o_ref):
    c = c_ref[0] + c_ref[1]
    o_ref[...] = jnp.where(c > 0.0, 1.0 / c, 0.0)


def _compute_inv(cpart):
    return pl.pallas_call(
        _inv_body,
        out_shape=jax.ShapeDtypeStruct((NSEGP // 128, 128), jnp.float32),
    )(cpart.reshape(2, NSEGP // 128, 128))


@functools.partial(
    pl.kernel,
    out_type=jax.ShapeDtypeStruct((2, NPAD, D), jnp.float32),
    mesh=_mesh,
    compiler_params=pltpu.CompilerParams(needs_layout_passes=False),
    scratch_types=[
        pltpu.VMEM_SHARED((NPAD, D), jnp.float32),
        pltpu.VMEM((CH,), jnp.int32),
        pltpu.VMEM((CH,), jnp.int32),
        pltpu.VMEM((CH,), jnp.int32),
        pltpu.VMEM((CH,), jnp.int32),
        pltpu.VMEM((CH,), jnp.int32),
        pltpu.VMEM((B,), jnp.int32),
        pltpu.VMEM((B,), jnp.int32),
        pltpu.VMEM((B,), jnp.float32),
        pltpu.VMEM((B,), jnp.float32),
        pltpu.VMEM((B, D), jnp.float32),
        pltpu.VMEM((B, D), jnp.float32),
        pltpu.SemaphoreType.DMA,
        pltpu.SemaphoreType.DMA,
        pltpu.SemaphoreType.DMA,
        pltpu.SemaphoreType.DMA,
        pltpu.SemaphoreType.DMA,
        pltpu.SemaphoreType.DMA,
    ],
)
def _sc_main(rowp, colp, typp, u2d, invh, out, acc_sp, rowc, colc, typc,
             segc, gixc, dstA, dstB, invA, invB, ubufA, ubufB,
             semuA, semiA, semaA, semuB, semiB, semaB):
    cid = lax.axis_index("c")
    sid = lax.axis_index("s")
    wid = cid * 16 + sid
    zrow = jnp.zeros((16,), jnp.float32)

    def zub(r, _):
        for k in range(8):
            ubufA[r, pl.ds(k * 16, 16)] = zrow
        return 0
    lax.fori_loop(0, B, zub, 0)

    def zacc(q, _):
        pltpu.sync_copy(ubufA, acc_sp.at[pl.ds(sid * 640 + q * B, B)])
        return 0
    lax.fori_loop(0, 5, zacc, 0)
    plsc.subcore_barrier()

    def issue(s, ubuf_, invb_, dstb_, semu, semi):
        o = s * B
        pltpu.async_copy(u2d.at[gixc.at[pl.ds(o, B)]], ubuf_, semu)
        pltpu.async_copy(invh.at[segc.at[pl.ds(o, B)]], invb_, semi)
        for p in range(8):
            dv = rowc[pl.ds(o + p * 16, 16)]
            dstb_[pl.ds(p * 16, 16)] = dv

    def wait_gather(ubuf_, invb_, semu, semi):
        pltpu.make_async_copy(u2d.at[gixc.at[pl.ds(0, B)]], ubuf_, semu).wait()
        pltpu.make_async_copy(invh.at[segc.at[pl.ds(0, B)]], invb_, semi).wait()

    def scale(ubuf_, invb_):
        def e_body(e, _):
            sv = plsc.load_gather(invb_, [jnp.full((16,), e, jnp.int32)])
            for k in range(8):
                sl = pl.ds(k * 16, 16)
                ubuf_[e, sl] = ubuf_[e, sl] * sv
            return 0
        lax.fori_loop(0, B, e_body, 0)

    def issue_add(ubuf_, dstb_, sema):
        pltpu.async_copy(ubuf_, acc_sp.at[dstb_], sema, add=True)

    def wait_add(ubuf_, dstb_, sema):
        pltpu.make_async_copy(ubuf_, acc_sp.at[dstb_], sema).wait()

    ew = jnp.where(cid == 0, EW0, EW1)
    ebase = cid * (16 * EW0) + sid * ew

    def chunk(c, _):
        base = ebase + c * CH
        pltpu.sync_copy(rowp.at[pl.ds(base, CH)], rowc)
        pltpu.sync_copy(colp.at[pl.ds(base, CH)], colc)
        pltpu.sync_copy(typp.at[pl.ds(base, CH)], typc)

        def group(g, _):
            sl = pl.ds(g * 16, 16)
            r16 = rowc[sl]
            t16 = typc[sl]
            segc[sl] = r16 * 7 + t16
            gixc[sl] = t16 * N + colc[sl]
            return 0
        lax.fori_loop(0, CH // 16, group, 0)

        issue(0, ubufA, invA, dstA, semuA, semiA)
        issue(1, ubufB, invB, dstB, semuB, semiB)

        def pair(q, _):
            wait_gather(ubufA, invA, semuA, semiA)
            scale(ubufA, invA)
            issue_add(ubufA, dstA, semaA)
            wait_gather(ubufB, invB, semuB, semiB)
            scale(ubufB, invB)
            issue_add(ubufB, dstB, semaB)
            wait_add(ubufA, dstA, semaA)
            issue(2 * q + 2, ubufA, invA, dstA, semuA, semiA)
            wait_add(ubufB, dstB, semaB)
            issue(2 * q + 3, ubufB, invB, dstB, semuB, semiB)
            return 0
        lax.fori_loop(0, NB // 2 - 1, pair, 0)

        wait_gather(ubufA, invA, semuA, semiA)
        scale(ubufA, invA)
        issue_add(ubufA, dstA, semaA)
        wait_gather(ubufB, invB, semuB, semiB)
        scale(ubufB, invB)
        issue_add(ubufB, dstB, semaB)
        wait_add(ubufA, dstA, semaA)
        wait_add(ubufB, dstB, semaB)
        return 0
    lax.fori_loop(0, ew // CH, chunk, 0)
    plsc.subcore_barrier()

    def wout(q, _):
        r0 = sid * 640 + q * B
        pltpu.sync_copy(acc_sp.at[pl.ds(r0, B)], out.at[cid, pl.ds(r0, B)])
        return 0
    lax.fori_loop(0, 5, wout, 0)


def _add_body(p_ref, o_ref):
    o_ref[...] = p_ref[0] + p_ref[1]


def _combine(p):
    return pl.pallas_call(
        _add_body,
        grid=(5,),
        in_specs=[pl.BlockSpec((2, N // 5, D), lambda j: (0, j, 0))],
        out_specs=pl.BlockSpec((N // 5, D), lambda j: (j, 0)),
        out_shape=jax.ShapeDtypeStruct((N, D), jnp.float32),
    )(p)


def kernel(x, edge_index, edge_type, weights):
    row = edge_index[0].astype(jnp.int32)
    col = edge_index[1].astype(jnp.int32)
    typ = edge_type.astype(jnp.int32)
    pad = EPAD - E
    rowp = jnp.concatenate([row, jnp.full((pad,), N, jnp.int32)])
    colp = jnp.concatenate([col, jnp.zeros((pad,), jnp.int32)])
    typp = jnp.concatenate([typ, jnp.zeros((pad,), jnp.int32)])
    w3 = weights.reshape(T, D, D)

    u2d = _compute_u(x, w3).reshape(NSEG, D)
    cpart = _sc_counts(rowp, typp)
    inv = _compute_inv(cpart).reshape(NSEGP)
    part = _sc_main(rowp, colp, typp, u2d, inv)
    return _combine(part[:, :N, :])


# speedup vs baseline: 1.1963x; 1.0352x over previous
import functools

import jax
import jax.numpy as jnp
from jax import lax
from jax.experimental import pallas as pl
from jax.experimental.pallas import tpu as pltpu
from jax.experimental.pallas import tpu_sc as plsc

N = 10000
D = 128
T = 7
NSEG = N * T
NSEGP = 71680
E = 320000
EPAD = 327680
NW = 32
EW = EPAD // NW
B = 128
CH = 1024
NB = CH // B
NPAD = 10240
EW0 = 15360
EW1 = 5120

_mesh = plsc.VectorSubcoreMesh(core_axis_name="c", subcore_axis_name="s")


def _iota16():
    return lax.broadcasted_iota(jnp.int32, (16,), 0)


def _mm_body(x_ref, w_ref, u_ref):
    u_ref[0] = jnp.dot(x_ref[...], w_ref[0], preferred_element_type=jnp.float32)


def _compute_u(x, w3):
    return pl.pallas_call(
        _mm_body,
        grid=(T, 5),
        in_specs=[
            pl.BlockSpec((N // 5, D), lambda t, j: (j, 0)),
            pl.BlockSpec((1, D, D), lambda t, j: (t, 0, 0)),
        ],
        out_specs=pl.BlockSpec((1, N // 5, D), lambda t, j: (t, j, 0)),
        out_shape=jax.ShapeDtypeStruct((T, N, D), jnp.float32),
    )(x, w3)


@functools.partial(
    pl.kernel,
    out_type=jax.ShapeDtypeStruct((2 * NSEGP,), jnp.float32),
    mesh=_mesh,
    compiler_params=pltpu.CompilerParams(needs_layout_passes=False),
    scratch_types=[
        pltpu.VMEM_SHARED((NSEGP,), jnp.float32),
        pltpu.VMEM((B,), jnp.float32),
        pltpu.VMEM((CH,), jnp.int32),
        pltpu.VMEM((CH,), jnp.int32),
        pltpu.VMEM((CH,), jnp.int32),
        pltpu.VMEM((2240,), jnp.float32),
        pltpu.SemaphoreType.DMA,
    ],
)
def _sc_counts(rowp, typp, cnt_out, counts_sp, ones, rowc, typc, segc,
               zchunk, semc):
    cid = lax.axis_index("c")
    sid = lax.axis_index("s")
    wid = cid * 16 + sid
    one_v = jnp.full((16,), 1.0, jnp.float32)
    zero_v = jnp.zeros((16,), jnp.float32)

    def fill(r, _):
        ones[pl.ds(r * 16, 16)] = one_v
        return 0
    lax.fori_loop(0, B // 16, fill, 0)

    def zfill(r, _):
        zchunk[pl.ds(r * 16, 16)] = zero_v
        return 0
    lax.fori_loop(0, 140, zfill, 0)

    def zslice(q, _):
        pltpu.sync_copy(zchunk, counts_sp.at[pl.ds(sid * 4480 + q * 2240, 2240)])
        return 0
    lax.fori_loop(0, 2, zslice, 0)
    plsc.subcore_barrier()

    def chunk(c, _):
        base = wid * EW + c * CH
        pltpu.sync_copy(rowp.at[pl.ds(base, CH)], rowc)
        pltpu.sync_copy(typp.at[pl.ds(base, CH)], typc)

        def group(g, _):
            sl = pl.ds(g * 16, 16)
            segc[sl] = rowc[sl] * 7 + typc[sl]
            return 0
        lax.fori_loop(0, CH // 16, group, 0)
        for s in range(NB):
            pltpu.async_copy(
                ones, counts_sp.at[segc.at[pl.ds(s * B, B)]], semc, add=True)
        for s in range(NB):
            pltpu.make_async_copy(
                ones, counts_sp.at[segc.at[pl.ds(0, B)]], semc).wait()
        return 0
    lax.fori_loop(0, EW // CH, chunk, 0)
    plsc.subcore_barrier()

    def rb(q, _):
        r0 = sid * 4480 + q * 2240
        pltpu.sync_copy(counts_sp.at[pl.ds(r0, 2240)], zchunk)
        pltpu.sync_copy(zchunk, cnt_out.at[pl.ds(cid * NSEGP + r0, 2240)])
        return 0
    lax.fori_loop(0, 2, rb, 0)


def _inv_body(c_ref, o_ref):
    c = c_ref[0] + c_ref[1]
    o_ref[...] = jnp.where(c > 0.0, 1.0 / c, 0.0)


def _compute_inv(cpart):
    return pl.pallas_call(
        _inv_body,
        out_shape=jax.ShapeDtypeStruct((NSEGP // 128, 128), jnp.float32),
    )(cpart.reshape(2, NSEGP // 128, 128))


@functools.partial(
    pl.kernel,
    out_type=jax.ShapeDtypeStruct((2, NPAD, D), jnp.float32),
    mesh=_mesh,
    compiler_params=pltpu.CompilerParams(needs_layout_passes=False),
    scratch_types=[
        pltpu.VMEM_SHARED((NPAD, D), jnp.float32),
        pltpu.VMEM((CH,), jnp.int32),
        pltpu.VMEM((CH,), jnp.int32),
        pltpu.VMEM((CH,), jnp.int32),
        pltpu.VMEM((CH,), jnp.int32),
        pltpu.VMEM((CH,), jnp.int32),
        pltpu.VMEM((B,), jnp.int32),
        pltpu.VMEM((B,), jnp.int32),
        pltpu.VMEM((B,), jnp.float32),
        pltpu.VMEM((B,), jnp.float32),
        pltpu.VMEM((B, D), jnp.float32),
        pltpu.VMEM((B, D), jnp.float32),
        pltpu.SemaphoreType.DMA,
        pltpu.SemaphoreType.DMA,
        pltpu.SemaphoreType.DMA,
        pltpu.SemaphoreType.DMA,
        pltpu.SemaphoreType.DMA,
        pltpu.SemaphoreType.DMA,
    ],
)
def _sc_main(rowp, colp, typp, u2d, invh, out, acc_sp, rowc, colc, typc,
             segc, gixc, dstA, dstB, invA, invB, ubufA, ubufB,
             semuA, semiA, semaA, semuB, semiB, semaB):
    cid = lax.axis_index("c")
    sid = lax.axis_index("s")
    wid = cid * 16 + sid
    zrow = jnp.zeros((16,), jnp.float32)

    def zub(r, _):
        for k in range(8):
            ubufA[r, pl.ds(k * 16, 16)] = zrow
        return 0
    lax.fori_loop(0, B, zub, 0)

    def zacc(q, _):
        pltpu.sync_copy(ubufA, acc_sp.at[pl.ds(sid * 640 + q * B, B)])
        return 0
    lax.fori_loop(0, 5, zacc, 0)
    plsc.subcore_barrier()

    def issue(s, ubuf_, invb_, dstb_, semu, semi):
        o = s * B
        pltpu.async_copy(u2d.at[gixc.at[pl.ds(o, B)]], ubuf_, semu)
        pltpu.async_copy(invh.at[segc.at[pl.ds(o, B)]], invb_, semi)
        for p in range(8):
            dv = rowc[pl.ds(o + p * 16, 16)]
            dstb_[pl.ds(p * 16, 16)] = dv

    def wait_gather(ubuf_, invb_, semu, semi):
        pltpu.make_async_copy(u2d.at[gixc.at[pl.ds(0, B)]], ubuf_, semu).wait()
        pltpu.make_async_copy(invh.at[segc.at[pl.ds(0, B)]], invb_, semi).wait()

    def scale(ubuf_, invb_):
        def e_body(e, _):
            sv = plsc.load_gather(invb_, [jnp.full((16,), e, jnp.int32)])
            for k in range(8):
                sl = pl.ds(k * 16, 16)
                ubuf_[e, sl] = ubuf_[e, sl] * sv
            return 0
        lax.fori_loop(0, B, e_body, 0)

    def issue_add(ubuf_, dstb_, sema):
        pltpu.async_copy(ubuf_, acc_sp.at[dstb_], sema, add=True)

    def wait_add(ubuf_, dstb_, sema):
        pltpu.make_async_copy(ubuf_, acc_sp.at[dstb_], sema).wait()

    ew = jnp.where(cid == 0, EW0, EW1)
    ebase = cid * (16 * EW0) + sid * ew

    def chunk(c, _):
        base = ebase + c * CH
        pltpu.sync_copy(rowp.at[pl.ds(base, CH)], rowc)
        pltpu.sync_copy(colp.at[pl.ds(base, CH)], colc)
        pltpu.sync_copy(typp.at[pl.ds(base, CH)], typc)

        def group(g, _):
            sl = pl.ds(g * 16, 16)
            r16 = rowc[sl]
            t16 = typc[sl]
            segc[sl] = r16 * 7 + t16
            gixc[sl] = t16 * N + colc[sl]
            return 0
        lax.fori_loop(0, CH // 16, group, 0)

        issue(0, ubufA, invA, dstA, semuA, semiA)
        issue(1, ubufB, invB, dstB, semuB, semiB)

        def pair(q, _):
            wait_gather(ubufA, invA, semuA, semiA)
            scale(ubufA, invA)
            issue_add(ubufA, dstA, semaA)
            wait_gather(ubufB, invB, semuB, semiB)
            scale(ubufB, invB)
            issue_add(ubufB, dstB, semaB)
            wait_add(ubufA, dstA, semaA)
            issue(2 * q + 2, ubufA, invA, dstA, semuA, semiA)
            wait_add(ubufB, dstB, semaB)
            issue(2 * q + 3, ubufB, invB, dstB, semuB, semiB)
            return 0
        lax.fori_loop(0, NB // 2 - 1, pair, 0)

        wait_gather(ubufA, invA, semuA, semiA)
        scale(ubufA, invA)
        issue_add(ubufA, dstA, semaA)
        wait_gather(ubufB, invB, semuB, semiB)
        scale(ubufB, invB)
        issue_add(ubufB, dstB, semaB)
        wait_add(ubufA, dstA, semaA)
        wait_add(ubufB, dstB, semaB)
        return 0
    lax.fori_loop(0, ew // CH, chunk, 0)
    plsc.subcore_barrier()

    def wout(q, _):
        r0 = sid * 640 + q * B
        pltpu.sync_copy(acc_sp.at[pl.ds(r0, B)], out.at[cid, pl.ds(r0, B)])
        return 0
    lax.fori_loop(0, 5, wout, 0)


def _add_body(p_ref, o_ref):
    o_ref[...] = p_ref[0] + p_ref[1]


def _combine(p):
    return pl.pallas_call(
        _add_body,
        grid=(5,),
        in_specs=[pl.BlockSpec((2, N // 5, D), lambda j: (0, j, 0))],
        out_specs=pl.BlockSpec((N // 5, D), lambda j: (j, 0)),
        out_shape=jax.ShapeDtypeStruct((N, D), jnp.float32),
    )(p)


def kernel(x, edge_index, edge_type, weights):
    row = edge_index[0].astype(jnp.int32)
    col = edge_index[1].astype(jnp.int32)
    typ = edge_type.astype(jnp.int32)
    pad = EPAD - E
    rowp = jnp.concatenate([row, jnp.full((pad,), N, jnp.int32)])
    colp = jnp.concatenate([col, jnp.zeros((pad,), jnp.int32)])
    typp = jnp.concatenate([typ, jnp.zeros((pad,), jnp.int32)])
    w3 = weights.reshape(T, D, D)

    u2d = _compute_u(x, w3).reshape(NSEG, D)
    cpart = _sc_counts(rowp, typp)
    inv = _compute_inv(cpart).reshape(NSEGP)
    part = _sc_main(rowp, colp, typp, u2d, inv)
    return _combine(part[:, :N, :])


# speedup vs baseline: 1.2407x; 1.0371x over previous
import functools

import jax
import jax.numpy as jnp
from jax import lax
from jax.experimental import pallas as pl
from jax.experimental.pallas import tpu as pltpu
from jax.experimental.pallas import tpu_sc as plsc

N = 10000
D = 128
T = 7
NSEG = N * T
NSEGP = 71680
E = 320000
EPAD = 327680
NW = 32
EW = EPAD // NW
B = 128
CH = 1024
NB = CH // B
NPAD = 10240
EW0 = 16384
EW1 = 4096

_mesh = plsc.VectorSubcoreMesh(core_axis_name="c", subcore_axis_name="s")


def _iota16():
    return lax.broadcasted_iota(jnp.int32, (16,), 0)


def _mm_body(x_ref, w_ref, u_ref):
    u_ref[0] = jnp.dot(x_ref[...], w_ref[0], preferred_element_type=jnp.float32)


def _compute_u(x, w3):
    return pl.pallas_call(
        _mm_body,
        grid=(T, 5),
        in_specs=[
            pl.BlockSpec((N // 5, D), lambda t, j: (j, 0)),
            pl.BlockSpec((1, D, D), lambda t, j: (t, 0, 0)),
        ],
        out_specs=pl.BlockSpec((1, N // 5, D), lambda t, j: (t, j, 0)),
        out_shape=jax.ShapeDtypeStruct((T, N, D), jnp.float32),
    )(x, w3)


@functools.partial(
    pl.kernel,
    out_type=jax.ShapeDtypeStruct((2 * NSEGP,), jnp.float32),
    mesh=_mesh,
    compiler_params=pltpu.CompilerParams(needs_layout_passes=False),
    scratch_types=[
        pltpu.VMEM_SHARED((NSEGP,), jnp.float32),
        pltpu.VMEM((B,), jnp.float32),
        pltpu.VMEM((CH,), jnp.int32),
        pltpu.VMEM((CH,), jnp.int32),
        pltpu.VMEM((CH,), jnp.int32),
        pltpu.VMEM((2240,), jnp.float32),
        pltpu.SemaphoreType.DMA,
    ],
)
def _sc_counts(rowp, typp, cnt_out, counts_sp, ones, rowc, typc, segc,
               zchunk, semc):
    cid = lax.axis_index("c")
    sid = lax.axis_index("s")
    wid = cid * 16 + sid
    one_v = jnp.full((16,), 1.0, jnp.float32)
    zero_v = jnp.zeros((16,), jnp.float32)

    def fill(r, _):
        ones[pl.ds(r * 16, 16)] = one_v
        return 0
    lax.fori_loop(0, B // 16, fill, 0)

    def zfill(r, _):
        zchunk[pl.ds(r * 16, 16)] = zero_v
        return 0
    lax.fori_loop(0, 140, zfill, 0)

    def zslice(q, _):
        pltpu.sync_copy(zchunk, counts_sp.at[pl.ds(sid * 4480 + q * 2240, 2240)])
        return 0
    lax.fori_loop(0, 2, zslice, 0)
    plsc.subcore_barrier()

    def chunk(c, _):
        base = wid * EW + c * CH
        pltpu.sync_copy(rowp.at[pl.ds(base, CH)], rowc)
        pltpu.sync_copy(typp.at[pl.ds(base, CH)], typc)

        def group(g, _):
            sl = pl.ds(g * 16, 16)
            segc[sl] = rowc[sl] * 7 + typc[sl]
            return 0
        lax.fori_loop(0, CH // 16, group, 0)
        for s in range(NB):
            pltpu.async_copy(
                ones, counts_sp.at[segc.at[pl.ds(s * B, B)]], semc, add=True)
        for s in range(NB):
            pltpu.make_async_copy(
                ones, counts_sp.at[segc.at[pl.ds(0, B)]], semc).wait()
        return 0
    lax.fori_loop(0, EW // CH, chunk, 0)
    plsc.subcore_barrier()

    def rb(q, _):
        r0 = sid * 4480 + q * 2240
        pltpu.sync_copy(counts_sp.at[pl.ds(r0, 2240)], zchunk)
        pltpu.sync_copy(zchunk, cnt_out.at[pl.ds(cid * NSEGP + r0, 2240)])
        return 0
    lax.fori_loop(0, 2, rb, 0)


def _inv_body(c_ref, o_ref):
    c = c_ref[0] + c_ref[1]
    o_ref[...] = jnp.where(c > 0.0, 1.0 / c, 0.0)


def _compute_inv(cpart):
    return pl.pallas_call(
        _inv_body,
        out_shape=jax.ShapeDtypeStruct((NSEGP // 128, 128), jnp.float32),
    )(cpart.reshape(2, NSEGP // 128, 128))


@functools.partial(
    pl.kernel,
    out_type=jax.ShapeDtypeStruct((2, NPAD, D), jnp.float32),
    mesh=_mesh,
    compiler_params=pltpu.CompilerParams(needs_layout_passes=False),
    scratch_types=[
        pltpu.VMEM_SHARED((NPAD, D), jnp.float32),
        pltpu.VMEM((CH,), jnp.int32),
        pltpu.VMEM((CH,), jnp.int32),
        pltpu.VMEM((CH,), jnp.int32),
        pltpu.VMEM((CH,), jnp.int32),
        pltpu.VMEM((CH,), jnp.int32),
        pltpu.VMEM((B,), jnp.int32),
        pltpu.VMEM((B,), jnp.int32),
        pltpu.VMEM((B,), jnp.float32),
        pltpu.VMEM((B,), jnp.float32),
        pltpu.VMEM((B, D), jnp.float32),
        pltpu.VMEM((B, D), jnp.float32),
        pltpu.SemaphoreType.DMA,
        pltpu.SemaphoreType.DMA,
        pltpu.SemaphoreType.DMA,
        pltpu.SemaphoreType.DMA,
        pltpu.SemaphoreType.DMA,
        pltpu.SemaphoreType.DMA,
    ],
)
def _sc_main(rowp, colp, typp, u2d, invh, out, acc_sp, rowc, colc, typc,
             segc, gixc, dstA, dstB, invA, invB, ubufA, ubufB,
             semuA, semiA, semaA, semuB, semiB, semaB):
    cid = lax.axis_index("c")
    sid = lax.axis_index("s")
    wid = cid * 16 + sid
    zrow = jnp.zeros((16,), jnp.float32)

    def zub(r, _):
        for k in range(8):
            ubufA[r, pl.ds(k * 16, 16)] = zrow
        return 0
    lax.fori_loop(0, B, zub, 0)

    def zacc(q, _):
        pltpu.sync_copy(ubufA, acc_sp.at[pl.ds(sid * 640 + q * B, B)])
        return 0
    lax.fori_loop(0, 5, zacc, 0)
    plsc.subcore_barrier()

    def issue(s, ubuf_, invb_, dstb_, semu, semi):
        o = s * B
        pltpu.async_copy(u2d.at[gixc.at[pl.ds(o, B)]], ubuf_, semu)
        pltpu.async_copy(invh.at[segc.at[pl.ds(o, B)]], invb_, semi)
        for p in range(8):
            dv = rowc[pl.ds(o + p * 16, 16)]
            dstb_[pl.ds(p * 16, 16)] = dv

    def wait_gather(ubuf_, invb_, semu, semi):
        pltpu.make_async_copy(u2d.at[gixc.at[pl.ds(0, B)]], ubuf_, semu).wait()
        pltpu.make_async_copy(invh.at[segc.at[pl.ds(0, B)]], invb_, semi).wait()

    def scale(ubuf_, invb_):
        def e_body(e, _):
            sv = plsc.load_gather(invb_, [jnp.full((16,), e, jnp.int32)])
            for k in range(8):
                sl = pl.ds(k * 16, 16)
                ubuf_[e, sl] = ubuf_[e, sl] * sv
            return 0
        lax.fori_loop(0, B, e_body, 0)

    def issue_add(ubuf_, dstb_, sema):
        pltpu.async_copy(ubuf_, acc_sp.at[dstb_], sema, add=True)

    def wait_add(ubuf_, dstb_, sema):
        pltpu.make_async_copy(ubuf_, acc_sp.at[dstb_], sema).wait()

    ew = jnp.where(cid == 0, EW0, EW1)
    ebase = cid * (16 * EW0) + sid * ew

    def chunk(c, _):
        base = ebase + c * CH
        pltpu.sync_copy(rowp.at[pl.ds(base, CH)], rowc)
        pltpu.sync_copy(colp.at[pl.ds(base, CH)], colc)
        pltpu.sync_copy(typp.at[pl.ds(base, CH)], typc)

        def group(g, _):
            sl = pl.ds(g * 16, 16)
            r16 = rowc[sl]
            t16 = typc[sl]
            segc[sl] = r16 * 7 + t16
            gixc[sl] = t16 * N + colc[sl]
            return 0
        lax.fori_loop(0, CH // 16, group, 0)

        issue(0, ubufA, invA, dstA, semuA, semiA)
        issue(1, ubufB, invB, dstB, semuB, semiB)

        def pair(q, _):
            wait_gather(ubufA, invA, semuA, semiA)
            scale(ubufA, invA)
            issue_add(ubufA, dstA, semaA)
            wait_gather(ubufB, invB, semuB, semiB)
            scale(ubufB, invB)
            issue_add(ubufB, dstB, semaB)
            wait_add(ubufA, dstA, semaA)
            issue(2 * q + 2, ubufA, invA, dstA, semuA, semiA)
            wait_add(ubufB, dstB, semaB)
            issue(2 * q + 3, ubufB, invB, dstB, semuB, semiB)
            return 0
        lax.fori_loop(0, NB // 2 - 1, pair, 0)

        wait_gather(ubufA, invA, semuA, semiA)
        scale(ubufA, invA)
        issue_add(ubufA, dstA, semaA)
        wait_gather(ubufB, invB, semuB, semiB)
        scale(ubufB, invB)
        issue_add(ubufB, dstB, semaB)
        wait_add(ubufA, dstA, semaA)
        wait_add(ubufB, dstB, semaB)
        return 0
    lax.fori_loop(0, ew // CH, chunk, 0)
    plsc.subcore_barrier()

    def wout(q, _):
        r0 = sid * 640 + q * B
        pltpu.sync_copy(acc_sp.at[pl.ds(r0, B)], out.at[cid, pl.ds(r0, B)])
        return 0
    lax.fori_loop(0, 5, wout, 0)


def _add_body(p_ref, o_ref):
    o_ref[...] = p_ref[0] + p_ref[1]


def _combine(p):
    return pl.pallas_call(
        _add_body,
        grid=(5,),
        in_specs=[pl.BlockSpec((2, N // 5, D), lambda j: (0, j, 0))],
        out_specs=pl.BlockSpec((N // 5, D), lambda j: (j, 0)),
        out_shape=jax.ShapeDtypeStruct((N, D), jnp.float32),
    )(p)


def kernel(x, edge_index, edge_type, weights):
    row = edge_index[0].astype(jnp.int32)
    col = edge_index[1].astype(jnp.int32)
    typ = edge_type.astype(jnp.int32)
    pad = EPAD - E
    rowp = jnp.concatenate([row, jnp.full((pad,), N, jnp.int32)])
    colp = jnp.concatenate([col, jnp.zeros((pad,), jnp.int32)])
    typp = jnp.concatenate([typ, jnp.zeros((pad,), jnp.int32)])
    w3 = weights.reshape(T, D, D)

    u2d = _compute_u(x, w3).reshape(NSEG, D)
    cpart = _sc_counts(rowp, typp)
    inv = _compute_inv(cpart).reshape(NSEGP)
    part = _sc_main(rowp, colp, typp, u2d, inv)
    return _combine(part[:, :N, :])


# speedup vs baseline: 1.4855x; 1.1973x over previous
import functools

import jax
import jax.numpy as jnp
from jax import lax
from jax.experimental import pallas as pl
from jax.experimental.pallas import tpu as pltpu
from jax.experimental.pallas import tpu_sc as plsc

N = 10000
D = 128
T = 7
NSEG = N * T
NSEGP = 71680
E = 320000
EPAD = 327680
NW = 32
EW = EPAD // NW
B = 128
CH = 1024
NB = CH // B
NPAD = 10240
EW0 = 18432
EW1 = 2048

_mesh = plsc.VectorSubcoreMesh(core_axis_name="c", subcore_axis_name="s")


def _iota16():
    return lax.broadcasted_iota(jnp.int32, (16,), 0)


def _mm_body(x_ref, w_ref, u_ref):
    u_ref[0] = jnp.dot(x_ref[...], w_ref[0], preferred_element_type=jnp.float32)


def _compute_u(x, w3):
    return pl.pallas_call(
        _mm_body,
        grid=(T, 5),
        in_specs=[
            pl.BlockSpec((N // 5, D), lambda t, j: (j, 0)),
            pl.BlockSpec((1, D, D), lambda t, j: (t, 0, 0)),
        ],
        out_specs=pl.BlockSpec((1, N // 5, D), lambda t, j: (t, j, 0)),
        out_shape=jax.ShapeDtypeStruct((T, N, D), jnp.float32),
    )(x, w3)


@functools.partial(
    pl.kernel,
    out_type=jax.ShapeDtypeStruct((2 * NSEGP,), jnp.float32),
    mesh=_mesh,
    compiler_params=pltpu.CompilerParams(needs_layout_passes=False),
    scratch_types=[
        pltpu.VMEM_SHARED((NSEGP,), jnp.float32),
        pltpu.VMEM((B,), jnp.float32),
        pltpu.VMEM((CH,), jnp.int32),
        pltpu.VMEM((CH,), jnp.int32),
        pltpu.VMEM((CH,), jnp.int32),
        pltpu.VMEM((2240,), jnp.float32),
        pltpu.SemaphoreType.DMA,
    ],
)
def _sc_counts(rowp, typp, cnt_out, counts_sp, ones, rowc, typc, segc,
               zchunk, semc):
    cid = lax.axis_index("c")
    sid = lax.axis_index("s")
    wid = cid * 16 + sid
    one_v = jnp.full((16,), 1.0, jnp.float32)
    zero_v = jnp.zeros((16,), jnp.float32)

    def fill(r, _):
        ones[pl.ds(r * 16, 16)] = one_v
        return 0
    lax.fori_loop(0, B // 16, fill, 0)

    def zfill(r, _):
        zchunk[pl.ds(r * 16, 16)] = zero_v
        return 0
    lax.fori_loop(0, 140, zfill, 0)

    def zslice(q, _):
        pltpu.sync_copy(zchunk, counts_sp.at[pl.ds(sid * 4480 + q * 2240, 2240)])
        return 0
    lax.fori_loop(0, 2, zslice, 0)
    plsc.subcore_barrier()

    def chunk(c, _):
        base = wid * EW + c * CH
        pltpu.sync_copy(rowp.at[pl.ds(base, CH)], rowc)
        pltpu.sync_copy(typp.at[pl.ds(base, CH)], typc)

        def group(g, _):
            sl = pl.ds(g * 16, 16)
            segc[sl] = rowc[sl] * 7 + typc[sl]
            return 0
        lax.fori_loop(0, CH // 16, group, 0)
        for s in range(NB):
            pltpu.async_copy(
                ones, counts_sp.at[segc.at[pl.ds(s * B, B)]], semc, add=True)
        for s in range(NB):
            pltpu.make_async_copy(
                ones, counts_sp.at[segc.at[pl.ds(0, B)]], semc).wait()
        return 0
    lax.fori_loop(0, EW // CH, chunk, 0)
    plsc.subcore_barrier()

    def rb(q, _):
        r0 = sid * 4480 + q * 2240
        pltpu.sync_copy(counts_sp.at[pl.ds(r0, 2240)], zchunk)
        pltpu.sync_copy(zchunk, cnt_out.at[pl.ds(cid * NSEGP + r0, 2240)])
        return 0
    lax.fori_loop(0, 2, rb, 0)


def _inv_body(c_ref, o_ref):
    c = c_ref[0] + c_ref[1]
    o_ref[...] = jnp.where(c > 0.0, 1.0 / c, 0.0)


def _compute_inv(cpart):
    return pl.pallas_call(
        _inv_body,
        out_shape=jax.ShapeDtypeStruct((NSEGP // 128, 128), jnp.float32),
    )(cpart.reshape(2, NSEGP // 128, 128))


@functools.partial(
    pl.kernel,
    out_type=jax.ShapeDtypeStruct((2, NPAD, D), jnp.float32),
    mesh=_mesh,
    compiler_params=pltpu.CompilerParams(needs_layout_passes=False),
    scratch_types=[
        pltpu.VMEM_SHARED((NPAD, D), jnp.float32),
        pltpu.VMEM((CH,), jnp.int32),
        pltpu.VMEM((CH,), jnp.int32),
        pltpu.VMEM((CH,), jnp.int32),
        pltpu.VMEM((CH,), jnp.int32),
        pltpu.VMEM((CH,), jnp.int32),
        pltpu.VMEM((B,), jnp.int32),
        pltpu.VMEM((B,), jnp.int32),
        pltpu.VMEM((B,), jnp.float32),
        pltpu.VMEM((B,), jnp.float32),
        pltpu.VMEM((B, D), jnp.float32),
        pltpu.VMEM((B, D), jnp.float32),
        pltpu.SemaphoreType.DMA,
        pltpu.SemaphoreType.DMA,
        pltpu.SemaphoreType.DMA,
        pltpu.SemaphoreType.DMA,
        pltpu.SemaphoreType.DMA,
        pltpu.SemaphoreType.DMA,
    ],
)
def _sc_main(rowp, colp, typp, u2d, invh, out, acc_sp, rowc, colc, typc,
             segc, gixc, dstA, dstB, invA, invB, ubufA, ubufB,
             semuA, semiA, semaA, semuB, semiB, semaB):
    cid = lax.axis_index("c")
    sid = lax.axis_index("s")
    wid = cid * 16 + sid
    zrow = jnp.zeros((16,), jnp.float32)

    def zub(r, _):
        for k in range(8):
            ubufA[r, pl.ds(k * 16, 16)] = zrow
        return 0
    lax.fori_loop(0, B, zub, 0)

    def zacc(q, _):
        pltpu.sync_copy(ubufA, acc_sp.at[pl.ds(sid * 640 + q * B, B)])
        return 0
    lax.fori_loop(0, 5, zacc, 0)
    plsc.subcore_barrier()

    def issue(s, ubuf_, invb_, dstb_, semu, semi):
        o = s * B
        pltpu.async_copy(u2d.at[gixc.at[pl.ds(o, B)]], ubuf_, semu)
        pltpu.async_copy(invh.at[segc.at[pl.ds(o, B)]], invb_, semi)
        for p in range(8):
            dv = rowc[pl.ds(o + p * 16, 16)]
            dstb_[pl.ds(p * 16, 16)] = dv

    def wait_gather(ubuf_, invb_, semu, semi):
        pltpu.make_async_copy(u2d.at[gixc.at[pl.ds(0, B)]], ubuf_, semu).wait()
        pltpu.make_async_copy(invh.at[segc.at[pl.ds(0, B)]], invb_, semi).wait()

    def scale(ubuf_, invb_):
        def e_body(e, _):
            sv = plsc.load_gather(invb_, [jnp.full((16,), e, jnp.int32)])
            for k in range(8):
                sl = pl.ds(k * 16, 16)
                ubuf_[e, sl] = ubuf_[e, sl] * sv
            return 0
        lax.fori_loop(0, B, e_body, 0)

    def issue_add(ubuf_, dstb_, sema):
        pltpu.async_copy(ubuf_, acc_sp.at[dstb_], sema, add=True)

    def wait_add(ubuf_, dstb_, sema):
        pltpu.make_async_copy(ubuf_, acc_sp.at[dstb_], sema).wait()

    ew = jnp.where(cid == 0, EW0, EW1)
    ebase = cid * (16 * EW0) + sid * ew

    def chunk(c, _):
        base = ebase + c * CH
        pltpu.sync_copy(rowp.at[pl.ds(base, CH)], rowc)
        pltpu.sync_copy(colp.at[pl.ds(base, CH)], colc)
        pltpu.sync_copy(typp.at[pl.ds(base, CH)], typc)

        def group(g, _):
            sl = pl.ds(g * 16, 16)
            r16 = rowc[sl]
            t16 = typc[sl]
            segc[sl] = r16 * 7 + t16
            gixc[sl] = t16 * N + colc[sl]
            return 0
        lax.fori_loop(0, CH // 16, group, 0)

        issue(0, ubufA, invA, dstA, semuA, semiA)
        issue(1, ubufB, invB, dstB, semuB, semiB)

        def pair(q, _):
            wait_gather(ubufA, invA, semuA, semiA)
            scale(ubufA, invA)
            issue_add(ubufA, dstA, semaA)
            wait_gather(ubufB, invB, semuB, semiB)
            scale(ubufB, invB)
            issue_add(ubufB, dstB, semaB)
            wait_add(ubufA, dstA, semaA)
            issue(2 * q + 2, ubufA, invA, dstA, semuA, semiA)
            wait_add(ubufB, dstB, semaB)
            issue(2 * q + 3, ubufB, invB, dstB, semuB, semiB)
            return 0
        lax.fori_loop(0, NB // 2 - 1, pair, 0)

        wait_gather(ubufA, invA, semuA, semiA)
        scale(ubufA, invA)
        issue_add(ubufA, dstA, semaA)
        wait_gather(ubufB, invB, semuB, semiB)
        scale(ubufB, invB)
        issue_add(ubufB, dstB, semaB)
        wait_add(ubufA, dstA, semaA)
        wait_add(ubufB, dstB, semaB)
        return 0
    lax.fori_loop(0, ew // CH, chunk, 0)
    plsc.subcore_barrier()

    def wout(q, _):
        r0 = sid * 640 + q * B
        pltpu.sync_copy(acc_sp.at[pl.ds(r0, B)], out.at[cid, pl.ds(r0, B)])
        return 0
    lax.fori_loop(0, 5, wout, 0)


def _add_body(p_ref, o_ref):
    o_ref[...] = p_ref[0] + p_ref[1]


def _combine(p):
    return pl.pallas_call(
        _add_body,
        grid=(5,),
        in_specs=[pl.BlockSpec((2, N // 5, D), lambda j: (0, j, 0))],
        out_specs=pl.BlockSpec((N // 5, D), lambda j: (j, 0)),
        out_shape=jax.ShapeDtypeStruct((N, D), jnp.float32),
    )(p)


def kernel(x, edge_index, edge_type, weights):
    row = edge_index[0].astype(jnp.int32)
    col = edge_index[1].astype(jnp.int32)
    typ = edge_type.astype(jnp.int32)
    pad = EPAD - E
    rowp = jnp.concatenate([row, jnp.full((pad,), N, jnp.int32)])
    colp = jnp.concatenate([col, jnp.zeros((pad,), jnp.int32)])
    typp = jnp.concatenate([typ, jnp.zeros((pad,), jnp.int32)])
    w3 = weights.reshape(T, D, D)

    u2d = _compute_u(x, w3).reshape(NSEG, D)
    cpart = _sc_counts(rowp, typp)
    inv = _compute_inv(cpart).reshape(NSEGP)
    part = _sc_main(rowp, colp, typp, u2d, inv)
    return _combine(part[:, :N, :])
